# Initial kernel scaffold; baseline (speedup 1.0000x reference)
import jax
import jax.numpy as jnp
from jax.experimental import pallas as pl


def kernel(x, edge_index, batch, W1_root, W1_nei, b1, W2_root, W2_nei, b2):
    raise NotImplementedError("write your pallas kernel here")


# speedup vs baseline: 4.6197x; 4.6197x over previous
import functools

import jax
import jax.numpy as jnp
from jax import lax
from jax.experimental import pallas as pl
from jax.experimental.pallas import tpu as pltpu
from jax.experimental.pallas import tpu_sc as plsc

N = 10000
E = 320000
D = 128
H = 128
G = 64

NC = 2
NS = 16
NW = NC * NS

CH = 128
K = 79
EPW = K * CH
EPAD = NW * EPW
NPAD = 79 * 128
CP = 632

BLK = 1000
NBLK = N // BLK


def _segsum_body(x_hbm, src_hbm, dst_hbm, out_hbm, src_v, dst_v, rows_v,
                 agg_s, sem):
    cid = lax.axis_index("c")
    sid = lax.axis_index("s")
    wid = sid * NC + cid

    zv = jnp.zeros((16,), jnp.float32)

    def zrow(r, _):
        for c8 in range(8):
            rows_v[r, pl.ds(c8 * 16, 16)] = zv
        return ()

    lax.fori_loop(0, CH, zrow, ())

    for t in range(5):
        c = sid + NS * t

        @pl.when(c < NPAD // 128)
        def _():
            base = pl.multiple_of(c * 128, 8)
            pltpu.sync_copy(rows_v, agg_s.at[pl.ds(base, 128)])

    plsc.subcore_barrier()

    pltpu.sync_copy(src_hbm.at[wid], src_v)
    pltpu.sync_copy(dst_hbm.at[wid], dst_v)

    def body(j, _):
        pltpu.async_copy(x_hbm.at[src_v.at[j]], rows_v, sem).wait()
        pltpu.sync_copy(rows_v, agg_s.at[dst_v.at[j]], add=True)
        return ()

    lax.fori_loop(0, K, body, ())

    plsc.subcore_barrier()
    obase = pl.multiple_of(sid * CP, 8)

    @pl.when(sid < NS - 1)
    def _():
        pltpu.sync_copy(agg_s.at[pl.ds(obase, CP)],
                        out_hbm.at[cid, pl.ds(obase, CP)])

    @pl.when(sid == NS - 1)
    def _():
        last = pl.multiple_of((NS - 1) * CP, 8)
        pltpu.sync_copy(agg_s.at[pl.ds(last, N - (NS - 1) * CP)],
                        out_hbm.at[cid, pl.ds(last, N - (NS - 1) * CP)])


_segsum = pl.kernel(
    _segsum_body,
    mesh=plsc.VectorSubcoreMesh(core_axis_name="c", subcore_axis_name="s"),
    out_type=jax.ShapeDtypeStruct((NC, N, D), jnp.float32),
    scratch_types=[
        pltpu.VMEM((K, CH), jnp.int32),
        pltpu.VMEM((K, CH), jnp.int32),
        pltpu.VMEM((CH, D), jnp.float32),
        pltpu.VMEM_SHARED((NPAD, D), jnp.float32),
        pltpu.SemaphoreType.DMA,
    ],
)


def _layer1_body(x_ref, p0_ref, p1_ref, wr_ref, wn_ref, b_ref, o_ref):
    agg = p0_ref[...] + p1_ref[...]
    acc = jnp.dot(x_ref[...], wr_ref[...], preferred_element_type=jnp.float32)
    acc = acc + jnp.dot(agg, wn_ref[...], preferred_element_type=jnp.float32)
    o_ref[...] = jnp.maximum(acc + b_ref[...], 0.0)


def _layer2_pool_body(h_ref, q0_ref, q1_ref, wr_ref, wn_ref, b_ref, batch_ref,
                      o_ref, acc_ref, cnt_ref):
    i = pl.program_id(0)

    @pl.when(i == 0)
    def _():
        acc_ref[...] = jnp.zeros_like(acc_ref)
        cnt_ref[...] = jnp.zeros_like(cnt_ref)

    agg = q0_ref[...] + q1_ref[...]
    h2 = jnp.dot(h_ref[...], wr_ref[...], preferred_element_type=jnp.float32)
    h2 = h2 + jnp.dot(agg, wn_ref[...], preferred_element_type=jnp.float32)
    h2 = jnp.maximum(h2 + b_ref[...], 0.0)

    bvec = batch_ref[0, 0, :]
    onehot = (bvec[:, None] == lax.broadcasted_iota(jnp.int32, (BLK, G), 1)
              ).astype(jnp.float32)
    acc_ref[...] += lax.dot_general(onehot, h2, (((0,), (0,)), ((), ())),
                                    preferred_element_type=jnp.float32)
    cnt_ref[...] += lax.dot_general(onehot, jnp.ones((BLK, H), jnp.float32),
                                    (((0,), (0,)), ((), ())),
                                    preferred_element_type=jnp.float32)

    @pl.when(i == pl.num_programs(0) - 1)
    def _():
        o_ref[...] = acc_ref[...] / jnp.maximum(cnt_ref[...], 1.0)


_layer1 = pl.pallas_call(
    _layer1_body,
    grid=(NBLK,),
    in_specs=[
        pl.BlockSpec((BLK, D), lambda i: (i, 0)),
        pl.BlockSpec((BLK, D), lambda i: (i, 0)),
        pl.BlockSpec((BLK, D), lambda i: (i, 0)),
        pl.BlockSpec((D, H), lambda i: (0, 0)),
        pl.BlockSpec((D, H), lambda i: (0, 0)),
        pl.BlockSpec((1, H), lambda i: (0, 0)),
    ],
    out_specs=pl.BlockSpec((BLK, H), lambda i: (i, 0)),
    out_shape=jax.ShapeDtypeStruct((N, H), jnp.float32),
)

_layer2_pool = pl.pallas_call(
    _layer2_pool_body,
    grid=(NBLK,),
    in_specs=[
        pl.BlockSpec((BLK, H), lambda i: (i, 0)),
        pl.BlockSpec((BLK, H), lambda i: (i, 0)),
        pl.BlockSpec((BLK, H), lambda i: (i, 0)),
        pl.BlockSpec((H, H), lambda i: (0, 0)),
        pl.BlockSpec((H, H), lambda i: (0, 0)),
        pl.BlockSpec((1, H), lambda i: (0, 0)),
        pl.BlockSpec((1, 1, BLK), lambda i: (i, 0, 0)),
    ],
    out_specs=pl.BlockSpec((G, H), lambda i: (0, 0)),
    out_shape=jax.ShapeDtypeStruct((G, H), jnp.float32),
    scratch_shapes=[
        pltpu.VMEM((G, H), jnp.float32),
        pltpu.VMEM((G, H), jnp.float32),
    ],
)


@jax.jit
def kernel(x, edge_index, batch, W1_root, W1_nei, b1, W2_root, W2_nei, b2):
    src = edge_index[0]
    dst = edge_index[1]
    pad = EPAD - E
    src_p = jnp.concatenate([src, jnp.zeros((pad,), jnp.int32)]
                            ).reshape(NW, K, CH)
    dst_p = jnp.concatenate([dst, jnp.full((pad,), N, jnp.int32)]
                            ).reshape(NW, K, CH)
    batch_r = batch.reshape(NBLK, 1, BLK)

    p = _segsum(x, src_p, dst_p)
    h = _layer1(x, p[0], p[1], W1_root, W1_nei, b1.reshape(1, H))
    q = _segsum(h, src_p, dst_p)
    out = _layer2_pool(h, q[0], q[1], W2_root, W2_nei, b2.reshape(1, H),
                       batch_r)
    return out
